# Initial kernel scaffold; baseline (speedup 1.0000x reference)
import jax
import jax.numpy as jnp
from jax.experimental import pallas as pl


def kernel(x, edge_attr, edge_index, bn0_g, bn0_b, We0, be0, eps0, W1_0, b1_0, mg0, mb0, W2_0, b2_0, bn1_g, bn1_b, We1, be1, eps1, W1_1, b1_1, mg1, mb1, W2_1, b2_1, Wc1, bc1, cg, cb, Wc2, bc2):
    raise NotImplementedError("write your pallas kernel here")


# speedup vs baseline: 2.8751x; 2.8751x over previous
import functools

import jax
import jax.numpy as jnp
from jax import lax
from jax.experimental import pallas as pl
from jax.experimental.pallas import tpu as pltpu
from jax.experimental.pallas import tpu_sc as plsc

N_NODES = 10000
N_EDGES = 320000
D = 128
D_EDGE = 16

NC = 2
NS = 16
NW = NC * NS
EPW = N_EDGES // NW
K = 80
SB = 25
NSB = EPW // (SB * K)
NP = 10240
RPT = NP // NS
NL = D // 16


def _leaky(v):
    return jnp.where(v > 0.0, v, 0.01 * v)


def _bn(h, g, b):
    mu = jnp.mean(h, axis=0, keepdims=True)
    var = jnp.mean((h - mu) ** 2, axis=0, keepdims=True)
    return (h - mu) * lax.rsqrt(var + 1e-5) * g + b


def _node_bn_body(x_ref, g_ref, b_ref, o_ref):
    o_ref[...] = _bn(x_ref[...], g_ref[...], b_ref[...])


def _node_bn(x, g, b):
    return pl.pallas_call(
        _node_bn_body,
        out_shape=jax.ShapeDtypeStruct((N_NODES, D), jnp.float32),
    )(x, g, b)


_BE = 4000


def _edge_proj_body(ea_ref, w0_ref, b0_ref, w1_ref, b1_ref, e0_ref, e1_ref):
    ea = ea_ref[...]
    e0_ref[...] = jnp.dot(ea, w0_ref[...], preferred_element_type=jnp.float32) + b0_ref[...]
    e1_ref[...] = jnp.dot(ea, w1_ref[...], preferred_element_type=jnp.float32) + b1_ref[...]


def _edge_proj(edge_attr, W0, b0, W1, b1):
    grid = (N_EDGES // _BE,)
    return pl.pallas_call(
        _edge_proj_body,
        grid=grid,
        in_specs=[
            pl.BlockSpec((_BE, D_EDGE), lambda i: (i, 0)),
            pl.BlockSpec((D_EDGE, D), lambda i: (0, 0)),
            pl.BlockSpec((1, D), lambda i: (0, 0)),
            pl.BlockSpec((D_EDGE, D), lambda i: (0, 0)),
            pl.BlockSpec((1, D), lambda i: (0, 0)),
        ],
        out_specs=[
            pl.BlockSpec((_BE, D), lambda i: (i, 0)),
            pl.BlockSpec((_BE, D), lambda i: (i, 0)),
        ],
        out_shape=[
            jax.ShapeDtypeStruct((N_EDGES, D), jnp.float32),
            jax.ShapeDtypeStruct((N_EDGES, D), jnp.float32),
        ],
    )(edge_attr, W0, b0, W1, b1)


def _dense_mid_body(hn_ref, p_ref, eps_ref, w1_ref, b1_ref, mg_ref, mb_ref,
                    w2_ref, b2_ref, g1_ref, bb1_ref, h_ref, ha_ref):
    z = eps_ref[0, 0] * hn_ref[...] + p_ref[0, :N_NODES] + p_ref[1, :N_NODES]
    t = jnp.dot(z, w1_ref[...], preferred_element_type=jnp.float32) + b1_ref[...]
    t = _leaky(_bn(t, mg_ref[...], mb_ref[...]))
    h = jnp.dot(t, w2_ref[...], preferred_element_type=jnp.float32) + b2_ref[...]
    h_ref[...] = h
    ha_ref[...] = _leaky(_bn(h, g1_ref[...], bb1_ref[...]))


def _dense_mid(hn, p, eps1p, W1, b1, mg, mb, W2, b2, g1, bb1):
    return pl.pallas_call(
        _dense_mid_body,
        out_shape=[
            jax.ShapeDtypeStruct((N_NODES, D), jnp.float32),
            jax.ShapeDtypeStruct((N_NODES, D), jnp.float32),
        ],
    )(hn, p, eps1p, W1, b1, mg, mb, W2, b2, g1, bb1)


def _dense_final_body(ha_ref, p_ref, h_ref, eps_ref, w1_ref, b1_ref, mg_ref,
                      mb_ref, w2_ref, b2_ref, wc1_ref, bc1_ref, cg_ref,
                      cb_ref, wc2_ref, bc2_ref, o_ref):
    z = eps_ref[0, 0] * ha_ref[...] + p_ref[0, :N_NODES] + p_ref[1, :N_NODES]
    t = jnp.dot(z, w1_ref[...], preferred_element_type=jnp.float32) + b1_ref[...]
    t = _leaky(_bn(t, mg_ref[...], mb_ref[...]))
    h2 = h_ref[...] + jnp.dot(t, w2_ref[...], preferred_element_type=jnp.float32) + b2_ref[...]
    c = jnp.dot(h2, wc1_ref[...], preferred_element_type=jnp.float32) + bc1_ref[...]
    c = _leaky(_bn(c, cg_ref[...], cb_ref[...]))
    o_ref[...] = jnp.sum(c * wc2_ref[...], axis=1, keepdims=True) + bc2_ref[0, 0]


def _dense_final(ha, p, h, eps1p, W1, b1, mg, mb, W2, b2, Wc1, bc1, cg, cb,
                 wc2_row, bc2):
    return pl.pallas_call(
        _dense_final_body,
        out_shape=jax.ShapeDtypeStruct((N_NODES, 1), jnp.float32),
    )(ha, p, h, eps1p, W1, b1, mg, mb, W2, b2, Wc1, bc1, cg, cb, wc2_row, bc2)


def _sc_msg_body(hn_hbm, e_hbm, src_hbm, dst_hbm, out_hbm,
                 acc_sh, src_v, dst_v, e_v, g_v, sem):
    cc = lax.axis_index("c")
    s = lax.axis_index("s")
    wid = cc * NS + s

    @pl.loop(0, K)
    def _zfill(i):
        for j in range(NL):
            e_v[i, pl.ds(j * 16, 16)] = jnp.zeros((16,), jnp.float32)

    @pl.loop(0, RPT // K)
    def _zacc(r):
        pltpu.sync_copy(e_v, acc_sh.at[pl.ds(s * RPT + r * K, K)])

    plsc.subcore_barrier()

    @pl.loop(0, NSB)
    def _super(sb):
        pltpu.sync_copy(src_hbm.at[wid, sb], src_v)
        pltpu.sync_copy(dst_hbm.at[wid, sb], dst_v)

        @pl.loop(0, SB)
        def _chunk(c):
            ebase = wid * EPW + (sb * SB + c) * K
            pltpu.sync_copy(e_hbm.at[pl.ds(ebase, K)], e_v)
            pltpu.async_copy(hn_hbm.at[src_v.at[c]], g_v, sem).wait()

            @pl.loop(0, K)
            def _rows(i):
                for j in range(NL):
                    sl = pl.ds(j * 16, 16)
                    e_v[i, sl] = jnp.maximum(e_v[i, sl] + g_v[i, sl], 0.0)

            pltpu.sync_copy(e_v, acc_sh.at[dst_v.at[c]], add=True)

    plsc.subcore_barrier()

    @pl.loop(0, 5)
    def _out(r):
        rows = RPT // 5
        base = s * RPT + r * rows
        pltpu.sync_copy(acc_sh.at[pl.ds(base, rows)],
                        out_hbm.at[cc, pl.ds(base, rows)])


@functools.lru_cache(maxsize=1)
def _sc_msg_kernel():
    return pl.kernel(
        _sc_msg_body,
        out_type=jax.ShapeDtypeStruct((NC, NP, D), jnp.float32),
        mesh=plsc.VectorSubcoreMesh(core_axis_name="c", subcore_axis_name="s",
                                    num_cores=NC, num_subcores=NS),
        scratch_types=[
            pltpu.VMEM_SHARED((NP, D), jnp.float32),
            pltpu.VMEM((SB, K), jnp.int32),
            pltpu.VMEM((SB, K), jnp.int32),
            pltpu.VMEM((K, D), jnp.float32),
            pltpu.VMEM((K, D), jnp.float32),
            pltpu.SemaphoreType.DMA,
        ],
    )


def _sc_msg(hn, e, src, dst):
    return _sc_msg_kernel()(hn, e, src, dst)


def kernel(x, edge_attr, edge_index, bn0_g, bn0_b, We0, be0, eps0, W1_0, b1_0,
           mg0, mb0, W2_0, b2_0, bn1_g, bn1_b, We1, be1, eps1, W1_1, b1_1,
           mg1, mb1, W2_1, b2_1, Wc1, bc1, cg, cb, Wc2, bc2):
    r = lambda v: v.reshape(1, D)
    src = edge_index[0].reshape(NW, NSB, SB, K)
    dst = edge_index[1].reshape(NW, NSB, SB, K)

    hn0 = _node_bn(x, r(bn0_g), r(bn0_b))
    e0, e1 = _edge_proj(edge_attr, We0, r(be0), We1, r(be1))

    p0 = _sc_msg(hn0, e0, src, dst)
    h, ha = _dense_mid(hn0, p0, (1.0 + eps0).reshape(1, 1), W1_0, r(b1_0),
                       r(mg0), r(mb0), W2_0, r(b2_0), r(bn1_g), r(bn1_b))

    p1 = _sc_msg(ha, e1, src, dst)
    out = _dense_final(ha, p1, h, (1.0 + eps1).reshape(1, 1), W1_1, r(b1_1),
                       r(mg1), r(mb1), W2_1, r(b2_1), Wc1, r(bc1), r(cg),
                       r(cb), Wc2.reshape(1, D), bc2.reshape(1, 1))
    return out.reshape(-1)


# speedup vs baseline: 3.9676x; 1.3800x over previous
import functools

import jax
import jax.numpy as jnp
from jax import lax
from jax.experimental import pallas as pl
from jax.experimental.pallas import tpu as pltpu
from jax.experimental.pallas import tpu_sc as plsc

N_NODES = 10000
N_EDGES = 320000
D = 128
D_EDGE = 16

NC = 2
NS = 16
NW = NC * NS
EPW = N_EDGES // NW
K = 40
SB = 50
NSB = EPW // (SB * K)
NP = 10240
RPT = NP // NS
NL = D // 16


def _leaky(v):
    return jnp.where(v > 0.0, v, 0.01 * v)


def _bn(h, g, b):
    mu = jnp.mean(h, axis=0, keepdims=True)
    var = jnp.mean((h - mu) ** 2, axis=0, keepdims=True)
    return (h - mu) * lax.rsqrt(var + 1e-5) * g + b


def _node_bn_body(x_ref, g_ref, b_ref, o_ref):
    o_ref[...] = _bn(x_ref[...], g_ref[...], b_ref[...])


def _node_bn(x, g, b):
    return pl.pallas_call(
        _node_bn_body,
        out_shape=jax.ShapeDtypeStruct((N_NODES, D), jnp.float32),
    )(x, g, b)


_BE = 4000


def _edge_proj_body(ea_ref, w0_ref, b0_ref, w1_ref, b1_ref, e0_ref, e1_ref):
    ea = ea_ref[...]
    e0_ref[...] = jnp.dot(ea, w0_ref[...], preferred_element_type=jnp.float32) + b0_ref[...]
    e1_ref[...] = jnp.dot(ea, w1_ref[...], preferred_element_type=jnp.float32) + b1_ref[...]


def _edge_proj(edge_attr, W0, b0, W1, b1):
    grid = (N_EDGES // _BE,)
    return pl.pallas_call(
        _edge_proj_body,
        grid=grid,
        in_specs=[
            pl.BlockSpec((_BE, D_EDGE), lambda i: (i, 0)),
            pl.BlockSpec((D_EDGE, D), lambda i: (0, 0)),
            pl.BlockSpec((1, D), lambda i: (0, 0)),
            pl.BlockSpec((D_EDGE, D), lambda i: (0, 0)),
            pl.BlockSpec((1, D), lambda i: (0, 0)),
        ],
        out_specs=[
            pl.BlockSpec((_BE, D), lambda i: (i, 0)),
            pl.BlockSpec((_BE, D), lambda i: (i, 0)),
        ],
        out_shape=[
            jax.ShapeDtypeStruct((N_EDGES, D), jnp.float32),
            jax.ShapeDtypeStruct((N_EDGES, D), jnp.float32),
        ],
    )(edge_attr, W0, b0, W1, b1)


def _dense_mid_body(hn_ref, p_ref, eps_ref, w1_ref, b1_ref, mg_ref, mb_ref,
                    w2_ref, b2_ref, g1_ref, bb1_ref, h_ref, ha_ref):
    z = eps_ref[0, 0] * hn_ref[...] + p_ref[0, :N_NODES] + p_ref[1, :N_NODES]
    t = jnp.dot(z, w1_ref[...], preferred_element_type=jnp.float32) + b1_ref[...]
    t = _leaky(_bn(t, mg_ref[...], mb_ref[...]))
    h = jnp.dot(t, w2_ref[...], preferred_element_type=jnp.float32) + b2_ref[...]
    h_ref[...] = h
    ha_ref[...] = _leaky(_bn(h, g1_ref[...], bb1_ref[...]))


def _dense_mid(hn, p, eps1p, W1, b1, mg, mb, W2, b2, g1, bb1):
    return pl.pallas_call(
        _dense_mid_body,
        out_shape=[
            jax.ShapeDtypeStruct((N_NODES, D), jnp.float32),
            jax.ShapeDtypeStruct((N_NODES, D), jnp.float32),
        ],
    )(hn, p, eps1p, W1, b1, mg, mb, W2, b2, g1, bb1)


def _dense_final_body(ha_ref, p_ref, h_ref, eps_ref, w1_ref, b1_ref, mg_ref,
                      mb_ref, w2_ref, b2_ref, wc1_ref, bc1_ref, cg_ref,
                      cb_ref, wc2_ref, bc2_ref, o_ref):
    z = eps_ref[0, 0] * ha_ref[...] + p_ref[0, :N_NODES] + p_ref[1, :N_NODES]
    t = jnp.dot(z, w1_ref[...], preferred_element_type=jnp.float32) + b1_ref[...]
    t = _leaky(_bn(t, mg_ref[...], mb_ref[...]))
    h2 = h_ref[...] + jnp.dot(t, w2_ref[...], preferred_element_type=jnp.float32) + b2_ref[...]
    c = jnp.dot(h2, wc1_ref[...], preferred_element_type=jnp.float32) + bc1_ref[...]
    c = _leaky(_bn(c, cg_ref[...], cb_ref[...]))
    o_ref[...] = jnp.sum(c * wc2_ref[...], axis=1, keepdims=True) + bc2_ref[0, 0]


def _dense_final(ha, p, h, eps1p, W1, b1, mg, mb, W2, b2, Wc1, bc1, cg, cb,
                 wc2_row, bc2):
    return pl.pallas_call(
        _dense_final_body,
        out_shape=jax.ShapeDtypeStruct((N_NODES, 1), jnp.float32),
    )(ha, p, h, eps1p, W1, b1, mg, mb, W2, b2, Wc1, bc1, cg, cb, wc2_row, bc2)


def _sc_msg_body(hn_hbm, e_hbm, src_hbm, dst_hbm, out_hbm,
                 acc_sh, src_v, dst_v, e_v, g_v,
                 esem0, esem1, gsem0, gsem1):
    cc = lax.axis_index("c")
    s = lax.axis_index("s")
    wid = cc * NS + s
    esems = (esem0, esem1)
    gsems = (gsem0, gsem1)

    @pl.loop(0, K)
    def _zfill(i):
        for j in range(NL):
            e_v[0, i, pl.ds(j * 16, 16)] = jnp.zeros((16,), jnp.float32)

    @pl.loop(0, RPT // K)
    def _zacc(r):
        pltpu.sync_copy(e_v.at[0], acc_sh.at[pl.ds(s * RPT + r * K, K)])

    plsc.subcore_barrier()

    def _issue(sb, c, b):
        ebase = wid * EPW + (sb * SB + c) * K
        pltpu.async_copy(e_hbm.at[pl.ds(ebase, K)], e_v.at[b], esems[b])
        pltpu.async_copy(hn_hbm.at[src_v.at[c]], g_v.at[b], gsems[b])

    def _drain(b, c):
        pltpu.make_async_copy(e_hbm.at[pl.ds(0, K)], e_v.at[b],
                              esems[b]).wait()
        pltpu.make_async_copy(hn_hbm.at[src_v.at[c]], g_v.at[b],
                              gsems[b]).wait()

    @pl.loop(0, NSB)
    def _super(sb):
        pltpu.sync_copy(src_hbm.at[wid, sb], src_v)
        pltpu.sync_copy(dst_hbm.at[wid, sb], dst_v)
        _issue(sb, 0, 0)

        @pl.loop(0, SB // 2)
        def _pair(p):
            for b in range(2):
                c = p * 2 + b
                _drain(b, c)

                @pl.when(c + 1 < SB)
                def _prefetch():
                    _issue(sb, c + 1, 1 - b)

                @pl.loop(0, K)
                def _rows(i):
                    for j in range(NL):
                        sl = pl.ds(j * 16, 16)
                        e_v[b, i, sl] = jnp.maximum(
                            e_v[b, i, sl] + g_v[b, i, sl], 0.0)

                pltpu.sync_copy(e_v.at[b], acc_sh.at[dst_v.at[c]], add=True)

    plsc.subcore_barrier()

    @pl.loop(0, 5)
    def _out(r):
        rows = RPT // 5
        base = s * RPT + r * rows
        pltpu.sync_copy(acc_sh.at[pl.ds(base, rows)],
                        out_hbm.at[cc, pl.ds(base, rows)])


@functools.lru_cache(maxsize=1)
def _sc_msg_kernel():
    return pl.kernel(
        _sc_msg_body,
        out_type=jax.ShapeDtypeStruct((NC, NP, D), jnp.float32),
        mesh=plsc.VectorSubcoreMesh(core_axis_name="c", subcore_axis_name="s",
                                    num_cores=NC, num_subcores=NS),
        scratch_types=[
            pltpu.VMEM_SHARED((NP, D), jnp.float32),
            pltpu.VMEM((SB, K), jnp.int32),
            pltpu.VMEM((SB, K), jnp.int32),
            pltpu.VMEM((2, K, D), jnp.float32),
            pltpu.VMEM((2, K, D), jnp.float32),
            pltpu.SemaphoreType.DMA,
            pltpu.SemaphoreType.DMA,
            pltpu.SemaphoreType.DMA,
            pltpu.SemaphoreType.DMA,
        ],
    )


def _sc_msg(hn, e, src, dst):
    return _sc_msg_kernel()(hn, e, src, dst)


def kernel(x, edge_attr, edge_index, bn0_g, bn0_b, We0, be0, eps0, W1_0, b1_0,
           mg0, mb0, W2_0, b2_0, bn1_g, bn1_b, We1, be1, eps1, W1_1, b1_1,
           mg1, mb1, W2_1, b2_1, Wc1, bc1, cg, cb, Wc2, bc2):
    r = lambda v: v.reshape(1, D)
    src = edge_index[0].reshape(NW, NSB, SB, K)
    dst = edge_index[1].reshape(NW, NSB, SB, K)

    hn0 = _node_bn(x, r(bn0_g), r(bn0_b))
    e0, e1 = _edge_proj(edge_attr, We0, r(be0), We1, r(be1))

    p0 = _sc_msg(hn0, e0, src, dst)
    h, ha = _dense_mid(hn0, p0, (1.0 + eps0).reshape(1, 1), W1_0, r(b1_0),
                       r(mg0), r(mb0), W2_0, r(b2_0), r(bn1_g), r(bn1_b))

    p1 = _sc_msg(ha, e1, src, dst)
    out = _dense_final(ha, p1, h, (1.0 + eps1).reshape(1, 1), W1_1, r(b1_1),
                       r(mg1), r(mb1), W2_1, r(b2_1), Wc1, r(bc1), r(cg),
                       r(cb), Wc2.reshape(1, D), bc2.reshape(1, 1))
    return out.reshape(-1)


# speedup vs baseline: 4.3771x; 1.1032x over previous
import functools

import jax
import jax.numpy as jnp
from jax import lax
from jax.experimental import pallas as pl
from jax.experimental.pallas import tpu as pltpu
from jax.experimental.pallas import tpu_sc as plsc

N_NODES = 10000
N_EDGES = 320000
D = 128
D_EDGE = 16

NC = 2
NS = 16
NW = NC * NS
EPW = N_EDGES // NW
K = 40
SB = 25
NSB = EPW // (SB * K)
NB = 3
NP = 10240
RPT = NP // NS
NL = D // 16


def _leaky(v):
    return jnp.where(v > 0.0, v, 0.01 * v)


def _bn(h, g, b):
    mu = jnp.mean(h, axis=0, keepdims=True)
    var = jnp.mean((h - mu) ** 2, axis=0, keepdims=True)
    return (h - mu) * lax.rsqrt(var + 1e-5) * g + b


def _node_bn_body(x_ref, g_ref, b_ref, o_ref):
    o_ref[...] = _bn(x_ref[...], g_ref[...], b_ref[...])


def _node_bn(x, g, b):
    return pl.pallas_call(
        _node_bn_body,
        out_shape=jax.ShapeDtypeStruct((N_NODES, D), jnp.float32),
    )(x, g, b)


_BE = 4000


def _edge_proj_body(ea_ref, w_ref, b_ref, e_ref):
    e_ref[...] = (
        jnp.dot(ea_ref[...], w_ref[...], preferred_element_type=jnp.float32)
        + b_ref[...]
    )


def _edge_proj(edge_attr, W, b):
    return pl.pallas_call(
        _edge_proj_body,
        grid=(N_EDGES // _BE,),
        in_specs=[
            pl.BlockSpec((_BE, D_EDGE), lambda i: (i, 0)),
            pl.BlockSpec((D_EDGE, D), lambda i: (0, 0)),
            pl.BlockSpec((1, D), lambda i: (0, 0)),
        ],
        out_specs=pl.BlockSpec((_BE, D), lambda i: (i, 0)),
        out_shape=jax.ShapeDtypeStruct((N_EDGES, D), jnp.float32),
    )(edge_attr, W, b)


def _dense_mid_body(hn_ref, p_ref, eps_ref, w1_ref, b1_ref, mg_ref, mb_ref,
                    w2_ref, b2_ref, g1_ref, bb1_ref, h_ref, ha_ref):
    z = eps_ref[0, 0] * hn_ref[...] + p_ref[0, :N_NODES] + p_ref[1, :N_NODES]
    t = jnp.dot(z, w1_ref[...], preferred_element_type=jnp.float32) + b1_ref[...]
    t = _leaky(_bn(t, mg_ref[...], mb_ref[...]))
    h = jnp.dot(t, w2_ref[...], preferred_element_type=jnp.float32) + b2_ref[...]
    h_ref[...] = h
    ha_ref[...] = _leaky(_bn(h, g1_ref[...], bb1_ref[...]))


def _dense_mid(hn, p, eps1p, W1, b1, mg, mb, W2, b2, g1, bb1):
    return pl.pallas_call(
        _dense_mid_body,
        out_shape=[
            jax.ShapeDtypeStruct((N_NODES, D), jnp.float32),
            jax.ShapeDtypeStruct((N_NODES, D), jnp.float32),
        ],
    )(hn, p, eps1p, W1, b1, mg, mb, W2, b2, g1, bb1)


def _dense_final_body(ha_ref, p_ref, h_ref, eps_ref, w1_ref, b1_ref, mg_ref,
                      mb_ref, w2_ref, b2_ref, wc1_ref, bc1_ref, cg_ref,
                      cb_ref, wc2_ref, bc2_ref, o_ref):
    z = eps_ref[0, 0] * ha_ref[...] + p_ref[0, :N_NODES] + p_ref[1, :N_NODES]
    t = jnp.dot(z, w1_ref[...], preferred_element_type=jnp.float32) + b1_ref[...]
    t = _leaky(_bn(t, mg_ref[...], mb_ref[...]))
    h2 = h_ref[...] + jnp.dot(t, w2_ref[...], preferred_element_type=jnp.float32) + b2_ref[...]
    c = jnp.dot(h2, wc1_ref[...], preferred_element_type=jnp.float32) + bc1_ref[...]
    c = _leaky(_bn(c, cg_ref[...], cb_ref[...]))
    o_ref[...] = jnp.sum(c * wc2_ref[...], axis=1, keepdims=True) + bc2_ref[0, 0]


def _dense_final(ha, p, h, eps1p, W1, b1, mg, mb, W2, b2, Wc1, bc1, cg, cb,
                 wc2_row, bc2):
    return pl.pallas_call(
        _dense_final_body,
        out_shape=jax.ShapeDtypeStruct((N_NODES, 1), jnp.float32),
    )(ha, p, h, eps1p, W1, b1, mg, mb, W2, b2, Wc1, bc1, cg, cb, wc2_row, bc2)


def _sc_msg_body(hn_hbm, e_hbm, src_hbm, dst_hbm, out_hbm,
                 acc_sh, src_v, dst_v, e_v, g_v,
                 esem0, esem1, esem2, gsem0, gsem1, gsem2,
                 ssem0, ssem1, ssem2):
    cc = lax.axis_index("c")
    s = lax.axis_index("s")
    wid = cc * NS + s
    esems = (esem0, esem1, esem2)
    gsems = (gsem0, gsem1, gsem2)
    ssems = (ssem0, ssem1, ssem2)

    @pl.loop(0, K)
    def _zfill(i):
        for j in range(NL):
            e_v[0, i, pl.ds(j * 16, 16)] = jnp.zeros((16,), jnp.float32)

    @pl.loop(0, RPT // K)
    def _zacc(r):
        pltpu.sync_copy(e_v.at[0], acc_sh.at[pl.ds(s * RPT + r * K, K)])

    plsc.subcore_barrier()

    def _issue(sb, c, b):
        ebase = wid * EPW + (sb * SB + c) * K
        pltpu.async_copy(e_hbm.at[pl.ds(ebase, K)], e_v.at[b], esems[b])
        pltpu.async_copy(hn_hbm.at[src_v.at[c]], g_v.at[b], gsems[b])

    def _body(c, b):
        pltpu.make_async_copy(e_hbm.at[pl.ds(0, K)], e_v.at[b],
                              esems[b]).wait()
        pltpu.make_async_copy(hn_hbm.at[src_v.at[c]], g_v.at[b],
                              gsems[b]).wait()

        @pl.loop(0, K)
        def _rows(i):
            for j in range(NL):
                sl = pl.ds(j * 16, 16)
                e_v[b, i, sl] = jnp.maximum(e_v[b, i, sl] + g_v[b, i, sl], 0.0)

        pltpu.async_copy(e_v.at[b], acc_sh.at[dst_v.at[c]], ssems[b],
                         add=True)

        @pl.when(c > 0)
        def _drain_prev():
            bp = (b + NB - 1) % NB
            pltpu.make_async_copy(e_v.at[bp], acc_sh.at[dst_v.at[c]],
                                  ssems[bp]).wait()

    @pl.loop(0, NSB)
    def _super(sb):
        pltpu.sync_copy(src_hbm.at[wid, sb], src_v)
        pltpu.sync_copy(dst_hbm.at[wid, sb], dst_v)
        _issue(sb, 0, 0)
        _issue(sb, 1, 1)

        @pl.loop(0, SB // NB)
        def _triple(t):
            for k in range(NB):
                c = t * NB + k
                _body(c, k)

                bn = (k + 2) % NB

                @pl.when(c + 2 < SB)
                def _prefetch():
                    _issue(sb, c + 2, bn)

        _body(SB - 1, (SB - 1) % NB)

        bl = (SB - 1) % NB
        pltpu.make_async_copy(e_v.at[bl], acc_sh.at[dst_v.at[SB - 1]],
                              ssems[bl]).wait()

    plsc.subcore_barrier()

    @pl.loop(0, 5)
    def _out(r):
        rows = RPT // 5
        base = s * RPT + r * rows
        pltpu.sync_copy(acc_sh.at[pl.ds(base, rows)],
                        out_hbm.at[cc, pl.ds(base, rows)])


@functools.lru_cache(maxsize=1)
def _sc_msg_kernel():
    return pl.kernel(
        _sc_msg_body,
        out_type=jax.ShapeDtypeStruct((NC, NP, D), jnp.float32),
        mesh=plsc.VectorSubcoreMesh(core_axis_name="c", subcore_axis_name="s",
                                    num_cores=NC, num_subcores=NS),
        scratch_types=[
            pltpu.VMEM_SHARED((NP, D), jnp.float32),
            pltpu.VMEM((SB, K), jnp.int32),
            pltpu.VMEM((SB, K), jnp.int32),
            pltpu.VMEM((NB, K, D), jnp.float32),
            pltpu.VMEM((NB, K, D), jnp.float32),
            pltpu.SemaphoreType.DMA,
            pltpu.SemaphoreType.DMA,
            pltpu.SemaphoreType.DMA,
            pltpu.SemaphoreType.DMA,
            pltpu.SemaphoreType.DMA,
            pltpu.SemaphoreType.DMA,
            pltpu.SemaphoreType.DMA,
            pltpu.SemaphoreType.DMA,
            pltpu.SemaphoreType.DMA,
        ],
    )


def _sc_msg(hn, e, src, dst):
    return _sc_msg_kernel()(hn, e, src, dst)


def kernel(x, edge_attr, edge_index, bn0_g, bn0_b, We0, be0, eps0, W1_0, b1_0,
           mg0, mb0, W2_0, b2_0, bn1_g, bn1_b, We1, be1, eps1, W1_1, b1_1,
           mg1, mb1, W2_1, b2_1, Wc1, bc1, cg, cb, Wc2, bc2):
    r = lambda v: v.reshape(1, D)
    src = edge_index[0].reshape(NW, NSB, SB, K)
    dst = edge_index[1].reshape(NW, NSB, SB, K)

    hn0 = _node_bn(x, r(bn0_g), r(bn0_b))
    e0 = _edge_proj(edge_attr, We0, r(be0))
    e1 = _edge_proj(edge_attr, We1, r(be1))

    p0 = _sc_msg(hn0, e0, src, dst)
    h, ha = _dense_mid(hn0, p0, (1.0 + eps0).reshape(1, 1), W1_0, r(b1_0),
                       r(mg0), r(mb0), W2_0, r(b2_0), r(bn1_g), r(bn1_b))

    p1 = _sc_msg(ha, e1, src, dst)
    out = _dense_final(ha, p1, h, (1.0 + eps1).reshape(1, 1), W1_1, r(b1_1),
                       r(mg1), r(mb1), W2_1, r(b2_1), Wc1, r(bc1), r(cg),
                       r(cb), Wc2.reshape(1, D), bc2.reshape(1, 1))
    return out.reshape(-1)
